# baseline (speedup 1.0000x reference)
import functools

import jax
import jax.numpy as jnp
from jax import lax
from jax.experimental import pallas as pl
from jax.experimental.pallas import tpu as pltpu
from jax.experimental.pallas import tpu_sc as plsc

NS = 16
NC = 2
LANES = 16


def _row_chunk(rows_per: int, max_rows: int = 640) -> int:
    for d in range(-(-rows_per // max_rows), rows_per + 1):
        if rows_per % d == 0:
            return rows_per // d
    return 1


def _make_sc_aggregate(N: int, E: int, Hh: int):
    assert E % NS == 0 and Hh == 32
    Eper = E // NS
    K = min(128, Eper)
    assert Eper % 8 == 0
    nblocks = -(-Eper // K)
    overlap = nblocks * K - Eper
    Npad = -(-N // (NS * 8)) * NS * 8
    rows_per = Npad // NS
    CHUNK = _row_chunk(rows_per, max_rows=256)
    assert CHUNK % 8 == 0
    nz = rows_per // CHUNK

    mesh = plsc.VectorSubcoreMesh(core_axis_name="c", subcore_axis_name="s",
                                  num_cores=NC, num_subcores=NS)

    @functools.partial(
        pl.kernel,
        out_type=jax.ShapeDtypeStruct((NC * Npad, Hh), jnp.float32),
        mesh=mesh,
        compiler_params=pltpu.CompilerParams(use_tc_tiling_on_sc=False),
        scratch_types=[
            pltpu.VMEM_SHARED((Npad, Hh), jnp.float32),
            pltpu.VMEM((K,), jnp.int32),
            pltpu.VMEM((K,), jnp.int32),
            pltpu.VMEM((K,), jnp.int32),
            pltpu.VMEM((K,), jnp.int32),
            pltpu.VMEM((K,), jnp.float32),
            pltpu.VMEM((K,), jnp.float32),
            pltpu.VMEM((K, Hh), jnp.float32),
            pltpu.VMEM((K, Hh), jnp.float32),
            pltpu.VMEM((K,), jnp.int32),
            pltpu.VMEM((K,), jnp.int32),
            pltpu.VMEM((2 * Hh,), jnp.float32),
            pltpu.VMEM((CHUNK, Hh), jnp.float32),
            pltpu.SemaphoreType.DMA,
            pltpu.SemaphoreType.DMA,
            pltpu.SemaphoreType.DMA,
            pltpu.SemaphoreType.DMA,
            pltpu.SemaphoreType.DMA,
            pltpu.SemaphoreType.DMA,
        ],
    )
    def sc_agg(h2_hbm, src_hbm, dst_hbm, dist_hbm, wb_hbm, agg_hbm,
               acc, idx0, idx1, dsx0, dsx1, dis0, dis1, msg0, msg1,
               snp0, snp1, wb_v, buf_v, sl0, sl1, sg0, sg1, ss0, ss1):
        IDX = (idx0, idx1)
        DSX = (dsx0, dsx1)
        DIS = (dis0, dis1)
        MSG = (msg0, msg1)
        SNP = (snp0, snp1)
        SL = (sl0, sl1)
        SG = (sg0, sg1)
        SS = (ss0, ss1)
        c = lax.axis_index("c")
        s = lax.axis_index("s")

        pltpu.sync_copy(wb_hbm.at[pl.ds(c * Hh, Hh)], wb_v.at[pl.ds(0, Hh)])
        pltpu.sync_copy(wb_hbm.at[pl.ds(NC * Hh + c * Hh, Hh)],
                        wb_v.at[pl.ds(Hh, Hh)])
        w_lo = wb_v[pl.ds(0, LANES)]
        w_hi = wb_v[pl.ds(LANES, LANES)]
        b_lo = wb_v[pl.ds(Hh, LANES)]
        b_hi = wb_v[pl.ds(Hh + LANES, LANES)]

        zvec = jnp.zeros((LANES,), jnp.float32)

        def zrow(r, _):
            buf_v[r, pl.ds(0, LANES)] = zvec
            buf_v[r, pl.ds(LANES, LANES)] = zvec
            return 0

        lax.fori_loop(0, CHUNK, zrow, 0)
        for k in range(nz):
            pltpu.sync_copy(buf_v, acc.at[pl.ds(s * rows_per + k * CHUNK, CHUNK)])
        plsc.subcore_barrier()

        row_off = c * N

        def base_of(b):
            return s * Eper + jnp.minimum(b * K, Eper - K)

        def start_loads(p, b):
            base = base_of(b)
            pltpu.async_copy(src_hbm.at[pl.ds(base, K)], IDX[p], SL[p])
            pltpu.async_copy(dst_hbm.at[pl.ds(base, K)], DSX[p], SL[p])
            pltpu.async_copy(dist_hbm.at[pl.ds(base, K)], DIS[p], SL[p])

        def wait_loads(p):
            pltpu.make_async_copy(src_hbm.at[pl.ds(0, K)], IDX[p], SL[p]).wait()
            pltpu.make_async_copy(dst_hbm.at[pl.ds(0, K)], DSX[p], SL[p]).wait()
            pltpu.make_async_copy(dist_hbm.at[pl.ds(0, K)], DIS[p], SL[p]).wait()

        def add_off(p):
            for i in range(K // LANES):
                sl = pl.ds(i * LANES, LANES)
                IDX[p][sl] = IDX[p][sl] + row_off

        def start_gather(p):
            pltpu.async_copy(h2_hbm.at[pl.ds(s * 1024, K)], MSG[p], SG[p])

        def wait_gather(p):
            pltpu.make_async_copy(h2_hbm.at[pl.ds(s * 1024, K)], MSG[p],
                                  SG[p]).wait()

        def start_scatter(p):
            for i in range(K // LANES):
                sl = pl.ds(i * LANES, LANES)
                SNP[p][sl] = DSX[p][sl]

        def wait_scatter(p):
            pass

        def compute(p, tail):
            msg_v, dist_v = MSG[p], DIS[p]

            def edge_grp(i, _):
                dvec = dist_v[pl.ds(i * LANES, LANES)]
                for j in range(LANES):
                    d = dvec[j]
                    r = i * LANES + j
                    lo = msg_v[r, pl.ds(0, LANES)]
                    hi = msg_v[r, pl.ds(LANES, LANES)]
                    msg_v[r, pl.ds(0, LANES)] = jnp.maximum(
                        lo + d * w_lo + b_lo, 0.0)
                    msg_v[r, pl.ds(LANES, LANES)] = jnp.maximum(
                        hi + d * w_hi + b_hi, 0.0)
                return 0

            lax.fori_loop(0, K // LANES, edge_grp, 0)
            if tail and overlap:
                for j in range(overlap):
                    msg_v[j, pl.ds(0, LANES)] = zvec
                    msg_v[j, pl.ds(LANES, LANES)] = zvec

        if nblocks < 4:
            def block(b, _):
                start_loads(0, b)
                wait_loads(0)
                add_off(0)
                start_gather(0)
                wait_gather(0)
                compute(0, False)

                @pl.when(b == nblocks - 1)
                def _():
                    if overlap:
                        for j in range(overlap):
                            msg0[j, pl.ds(0, LANES)] = zvec
                            msg0[j, pl.ds(LANES, LANES)] = zvec

                start_scatter(0)
                wait_scatter(0)
                return 0

            lax.fori_loop(0, nblocks, block, 0)
        else:
            M = (nblocks - 2) // 2 if nblocks % 2 == 0 else (nblocks - 3) // 2
            R = nblocks - 2 * M

            start_loads(0, 0)
            wait_loads(0)
            add_off(0)
            start_gather(0)
            start_loads(1, 1)

            def pair(g, _):
                wait_loads(1)
                add_off(1)

                @pl.when(g > 0)
                def _():
                    wait_scatter(1)

                start_gather(1)
                wait_gather(0)
                compute(0, False)
                start_scatter(0)
                start_loads(0, 2 * g + 2)
                wait_gather(1)
                wait_loads(0)
                add_off(0)
                wait_scatter(0)
                start_gather(0)
                start_loads(1, 2 * g + 3)
                compute(1, False)
                start_scatter(1)
                return 0

            lax.fori_loop(0, M, pair, 0)

            wait_gather(0)
            wait_loads(1)
            add_off(1)
            wait_scatter(1)
            start_gather(1)
            if R == 3:
                start_loads(0, nblocks - 1)
            compute(0, False)
            start_scatter(0)
            wait_gather(1)
            if R == 2:
                compute(1, True)
                start_scatter(1)
                wait_scatter(0)
                wait_scatter(1)
            else:
                wait_loads(0)
                add_off(0)
                wait_scatter(0)
                start_gather(0)
                compute(1, False)
                start_scatter(1)
                wait_gather(0)
                compute(0, True)
                start_scatter(0)
                wait_scatter(1)
                wait_scatter(0)

        plsc.subcore_barrier()

        for k in range(nz):
            r0 = s * rows_per + k * CHUNK
            pltpu.sync_copy(acc.at[pl.ds(r0, CHUNK)], buf_v)
            pltpu.sync_copy(buf_v, agg_hbm.at[pl.ds(c * Npad + r0, CHUNK)])

    return sc_agg


def _embed_body(x_ref, w_ref, b_ref, out_ref):
    h = jnp.dot(x_ref[...], w_ref[...], preferred_element_type=jnp.float32)
    h = h + b_ref[...]
    out_ref[0] = h[:, :32]
    out_ref[1] = h[:, 32:]


def _tc_embed(x, W_node, b_node, Nb):
    N, D = x.shape
    H = W_node.shape[1]
    grid = (N // Nb,)
    return pl.pallas_call(
        _embed_body,
        grid=grid,
        in_specs=[
            pl.BlockSpec((Nb, D), lambda i: (i, 0)),
            pl.BlockSpec((D, H), lambda i: (0, 0)),
            pl.BlockSpec((1, H), lambda i: (0, 0)),
        ],
        out_specs=pl.BlockSpec((2, Nb, H // 2), lambda i: (0, i, 0)),
        out_shape=jax.ShapeDtypeStruct((2, N, H // 2), jnp.float32),
    )(x, W_node, b_node.reshape(1, H))


def _update_body(h_ref, a_ref, w_ref, b_ref, s_ref, out_ref):
    h = jnp.concatenate([h_ref[0], h_ref[1]], axis=1)
    a = jnp.concatenate([a_ref[0], a_ref[1]], axis=1)
    u = jnp.dot(s_ref[0, 0] * h + a, w_ref[...],
                preferred_element_type=jnp.float32) + b_ref[...]
    u = jnp.maximum(u, 0.0) + h
    out_ref[0] = u[:, :32]
    out_ref[1] = u[:, 32:]


def _tc_update(h2, agg2, W, b, eps, Nb):
    _, N, Hh = h2.shape
    H = 2 * Hh
    grid = (N // Nb,)
    blk = pl.BlockSpec((2, Nb, Hh), lambda i: (0, i, 0))
    return pl.pallas_call(
        _update_body,
        grid=grid,
        in_specs=[
            blk, blk,
            pl.BlockSpec((H, H), lambda i: (0, 0)),
            pl.BlockSpec((1, H), lambda i: (0, 0)),
            pl.BlockSpec((1, 1), lambda i: (0, 0)),
        ],
        out_specs=blk,
        out_shape=jax.ShapeDtypeStruct((2, N, Hh), jnp.float32),
    )(h2, agg2, W, b.reshape(1, H), (1.0 + eps).reshape(1, 1))


def _final_body(h_ref, a_ref, w_ref, b_ref, s_ref, batch_ref,
                w1_ref, b1_ref, w2_ref, b2_ref, out_ref,
                pooled, cnt, *, NG, nsteps):
    i = pl.program_id(0)

    @pl.when(i == 0)
    def _():
        pooled[...] = jnp.zeros_like(pooled)
        cnt[...] = jnp.zeros_like(cnt)

    h = jnp.concatenate([h_ref[0], h_ref[1]], axis=1)
    a = jnp.concatenate([a_ref[0], a_ref[1]], axis=1)
    u = jnp.dot(s_ref[0, 0] * h + a, w_ref[...],
                preferred_element_type=jnp.float32) + b_ref[...]
    u = jnp.maximum(u, 0.0) + h

    gids = lax.broadcasted_iota(jnp.int32, (1, NG), 1)
    P = (batch_ref[...] == gids).astype(jnp.float32)
    pooled[...] += lax.dot_general(P, u, (((0,), (0,)), ((), ())),
                                   preferred_element_type=jnp.float32)
    cnt[...] += jnp.sum(P, axis=0, keepdims=True)

    @pl.when(i == nsteps - 1)
    def _():
        mean = pooled[...] / jnp.maximum(cnt[...], 1.0).T
        r = jnp.maximum(
            jnp.dot(mean, w1_ref[...], preferred_element_type=jnp.float32)
            + b1_ref[...], 0.0)
        out_ref[...] = (jnp.dot(r, w2_ref[...],
                                preferred_element_type=jnp.float32)
                        + b2_ref[...])


def _tc_final(h2, agg2, W, b, eps, batch, W_d1, b_d1, W_d2, b_d2, NG, Nb):
    _, N, Hh = h2.shape
    H = 2 * Hh
    Hd = W_d1.shape[1]
    nsteps = N // Nb
    blk = pl.BlockSpec((2, Nb, Hh), lambda i: (0, i, 0))
    body = functools.partial(_final_body, NG=NG, nsteps=nsteps)
    return pl.pallas_call(
        body,
        grid=(nsteps,),
        in_specs=[
            blk, blk,
            pl.BlockSpec((H, H), lambda i: (0, 0)),
            pl.BlockSpec((1, H), lambda i: (0, 0)),
            pl.BlockSpec((1, 1), lambda i: (0, 0)),
            pl.BlockSpec((Nb, 1), lambda i: (i, 0)),
            pl.BlockSpec((H, Hd), lambda i: (0, 0)),
            pl.BlockSpec((1, Hd), lambda i: (0, 0)),
            pl.BlockSpec((Hd, 1), lambda i: (0, 0)),
            pl.BlockSpec((1, 1), lambda i: (0, 0)),
        ],
        out_specs=pl.BlockSpec((NG, 1), lambda i: (0, 0)),
        out_shape=jax.ShapeDtypeStruct((NG, 1), jnp.float32),
        scratch_shapes=[
            pltpu.VMEM((NG, H), jnp.float32),
            pltpu.VMEM((1, NG), jnp.float32),
        ],
    )(h2, agg2, W, b.reshape(1, H), (1.0 + eps).reshape(1, 1),
      batch.reshape(N, 1), W_d1, b_d1.reshape(1, Hd), W_d2,
      b_d2.reshape(1, 1))


def kernel(x, edge_dist, edge_index, batch,
           W_node, b_node, W_edge, b_edge,
           W_c1, b_c1, eps1, W_c2, b_c2, eps2, W_c3, b_c3, eps3,
           W_d1, b_d1, W_d2, b_d2):
    N, _ = x.shape
    E = edge_dist.shape[0]
    H = W_node.shape[1]
    Hh = H // 2
    NG = 64
    Nb = 2000 if N % 2000 == 0 else N

    src = edge_index[0]
    dst = edge_index[1]
    wb = jnp.concatenate([W_edge[0], b_edge])

    sc_agg = _make_sc_aggregate(N, E, Hh)
    Npad = -(-N // (NS * 8)) * NS * 8

    h2 = _tc_embed(x, W_node, b_node, Nb)
    out = None
    for li, (W, b, eps) in enumerate(
            ((W_c1, b_c1, eps1), (W_c2, b_c2, eps2), (W_c3, b_c3, eps3))):
        agg = sc_agg(h2.reshape(2 * N, Hh), src, dst, edge_dist, wb)
        agg2 = agg.reshape(2, Npad, Hh)
        if li < 2:
            h2 = _tc_update(h2, agg2, W, b, eps, Nb)
        else:
            out = _tc_final(h2, agg2, W, b, eps, batch,
                            W_d1, b_d1, W_d2, b_d2, NG, Nb)
    return out


# speedup vs baseline: 1.2636x; 1.2636x over previous
import functools

import jax
import jax.numpy as jnp
from jax import lax
from jax.experimental import pallas as pl
from jax.experimental.pallas import tpu as pltpu
from jax.experimental.pallas import tpu_sc as plsc

NS = 16
NC = 2
LANES = 16


def _row_chunk(rows_per: int, max_rows: int = 640) -> int:
    for d in range(-(-rows_per // max_rows), rows_per + 1):
        if rows_per % d == 0:
            return rows_per // d
    return 1


def _make_sc_aggregate(N: int, E: int, Hh: int):
    assert E % NS == 0 and Hh == 32
    Eper = E // NS
    K = min(256, Eper)
    assert Eper % 8 == 0
    nblocks = -(-Eper // K)
    overlap = nblocks * K - Eper
    Npad = -(-N // (NS * 8)) * NS * 8
    rows_per = Npad // NS
    CHUNK = _row_chunk(rows_per, max_rows=256)
    assert CHUNK % 8 == 0
    nz = rows_per // CHUNK

    mesh = plsc.VectorSubcoreMesh(core_axis_name="c", subcore_axis_name="s",
                                  num_cores=NC, num_subcores=NS)

    @functools.partial(
        pl.kernel,
        out_type=jax.ShapeDtypeStruct((NC * Npad, Hh), jnp.float32),
        mesh=mesh,
        compiler_params=pltpu.CompilerParams(use_tc_tiling_on_sc=False),
        scratch_types=[
            pltpu.VMEM_SHARED((Npad, Hh), jnp.float32),
            pltpu.VMEM((K,), jnp.int32),
            pltpu.VMEM((K,), jnp.int32),
            pltpu.VMEM((K,), jnp.int32),
            pltpu.VMEM((K,), jnp.int32),
            pltpu.VMEM((K,), jnp.float32),
            pltpu.VMEM((K,), jnp.float32),
            pltpu.VMEM((K, Hh), jnp.float32),
            pltpu.VMEM((K, Hh), jnp.float32),
            pltpu.VMEM((K,), jnp.int32),
            pltpu.VMEM((K,), jnp.int32),
            pltpu.VMEM((2 * Hh,), jnp.float32),
            pltpu.VMEM((CHUNK, Hh), jnp.float32),
            pltpu.SemaphoreType.DMA,
            pltpu.SemaphoreType.DMA,
            pltpu.SemaphoreType.DMA,
            pltpu.SemaphoreType.DMA,
            pltpu.SemaphoreType.DMA,
            pltpu.SemaphoreType.DMA,
        ],
    )
    def sc_agg(h2_hbm, src_hbm, dst_hbm, dist_hbm, wb_hbm, agg_hbm,
               acc, idx0, idx1, dsx0, dsx1, dis0, dis1, msg0, msg1,
               snp0, snp1, wb_v, buf_v, sl0, sl1, sg0, sg1, ss0, ss1):
        IDX = (idx0, idx1)
        DSX = (dsx0, dsx1)
        DIS = (dis0, dis1)
        MSG = (msg0, msg1)
        SNP = (snp0, snp1)
        SL = (sl0, sl1)
        SG = (sg0, sg1)
        SS = (ss0, ss1)
        c = lax.axis_index("c")
        s = lax.axis_index("s")

        pltpu.sync_copy(wb_hbm.at[pl.ds(c * Hh, Hh)], wb_v.at[pl.ds(0, Hh)])
        pltpu.sync_copy(wb_hbm.at[pl.ds(NC * Hh + c * Hh, Hh)],
                        wb_v.at[pl.ds(Hh, Hh)])
        w_lo = wb_v[pl.ds(0, LANES)]
        w_hi = wb_v[pl.ds(LANES, LANES)]
        b_lo = wb_v[pl.ds(Hh, LANES)]
        b_hi = wb_v[pl.ds(Hh + LANES, LANES)]

        zvec = jnp.zeros((LANES,), jnp.float32)

        def zrow(r, _):
            buf_v[r, pl.ds(0, LANES)] = zvec
            buf_v[r, pl.ds(LANES, LANES)] = zvec
            return 0

        lax.fori_loop(0, CHUNK, zrow, 0)
        for k in range(nz):
            pltpu.sync_copy(buf_v, acc.at[pl.ds(s * rows_per + k * CHUNK, CHUNK)])
        plsc.subcore_barrier()

        row_off = c * N

        def base_of(b):
            return s * Eper + jnp.minimum(b * K, Eper - K)

        def start_loads(p, b):
            base = base_of(b)
            pltpu.async_copy(src_hbm.at[pl.ds(base, K)], IDX[p], SL[p])
            pltpu.async_copy(dst_hbm.at[pl.ds(base, K)], DSX[p], SL[p])
            pltpu.async_copy(dist_hbm.at[pl.ds(base, K)], DIS[p], SL[p])

        def wait_loads(p):
            pltpu.make_async_copy(src_hbm.at[pl.ds(0, K)], IDX[p], SL[p]).wait()
            pltpu.make_async_copy(dst_hbm.at[pl.ds(0, K)], DSX[p], SL[p]).wait()
            pltpu.make_async_copy(dist_hbm.at[pl.ds(0, K)], DIS[p], SL[p]).wait()

        def add_off(p):
            for i in range(K // LANES):
                sl = pl.ds(i * LANES, LANES)
                IDX[p][sl] = IDX[p][sl] + row_off

        def start_gather(p):
            pltpu.async_copy(h2_hbm.at[IDX[p]], MSG[p], SG[p])

        def wait_gather(p):
            pltpu.make_async_copy(h2_hbm.at[IDX[p]], MSG[p], SG[p]).wait()

        def start_scatter(p):
            for i in range(K // LANES):
                sl = pl.ds(i * LANES, LANES)
                SNP[p][sl] = DSX[p][sl]
            pltpu.async_copy(MSG[p], acc.at[SNP[p]], SS[p], add=True)

        def wait_scatter(p):
            pltpu.make_async_copy(MSG[p], acc.at[SNP[p]], SS[p]).wait()

        def compute(p, tail):
            msg_v, dist_v = MSG[p], DIS[p]

            def edge_grp(i, _):
                dvec = dist_v[pl.ds(i * LANES, LANES)]
                for j in range(LANES):
                    d = dvec[j]
                    r = i * LANES + j
                    lo = msg_v[r, pl.ds(0, LANES)]
                    hi = msg_v[r, pl.ds(LANES, LANES)]
                    msg_v[r, pl.ds(0, LANES)] = jnp.maximum(
                        lo + d * w_lo + b_lo, 0.0)
                    msg_v[r, pl.ds(LANES, LANES)] = jnp.maximum(
                        hi + d * w_hi + b_hi, 0.0)
                return 0

            lax.fori_loop(0, K // LANES, edge_grp, 0)
            if tail and overlap:
                for j in range(overlap):
                    msg_v[j, pl.ds(0, LANES)] = zvec
                    msg_v[j, pl.ds(LANES, LANES)] = zvec

        if nblocks < 4:
            def block(b, _):
                start_loads(0, b)
                wait_loads(0)
                add_off(0)
                start_gather(0)
                wait_gather(0)
                compute(0, False)

                @pl.when(b == nblocks - 1)
                def _():
                    if overlap:
                        for j in range(overlap):
                            msg0[j, pl.ds(0, LANES)] = zvec
                            msg0[j, pl.ds(LANES, LANES)] = zvec

                start_scatter(0)
                wait_scatter(0)
                return 0

            lax.fori_loop(0, nblocks, block, 0)
        else:
            M = (nblocks - 2) // 2 if nblocks % 2 == 0 else (nblocks - 3) // 2
            R = nblocks - 2 * M

            start_loads(0, 0)
            wait_loads(0)
            add_off(0)
            start_gather(0)
            start_loads(1, 1)

            def pair(g, _):
                wait_loads(1)
                add_off(1)

                @pl.when(g > 0)
                def _():
                    wait_scatter(1)

                start_gather(1)
                wait_gather(0)
                compute(0, False)
                start_scatter(0)
                start_loads(0, 2 * g + 2)
                wait_gather(1)
                wait_loads(0)
                add_off(0)
                wait_scatter(0)
                start_gather(0)
                start_loads(1, 2 * g + 3)
                compute(1, False)
                start_scatter(1)
                return 0

            lax.fori_loop(0, M, pair, 0)

            wait_gather(0)
            wait_loads(1)
            add_off(1)
            wait_scatter(1)
            start_gather(1)
            if R == 3:
                start_loads(0, nblocks - 1)
            compute(0, False)
            start_scatter(0)
            wait_gather(1)
            if R == 2:
                compute(1, True)
                start_scatter(1)
                wait_scatter(0)
                wait_scatter(1)
            else:
                wait_loads(0)
                add_off(0)
                wait_scatter(0)
                start_gather(0)
                compute(1, False)
                start_scatter(1)
                wait_gather(0)
                compute(0, True)
                start_scatter(0)
                wait_scatter(1)
                wait_scatter(0)

        plsc.subcore_barrier()

        for k in range(nz):
            r0 = s * rows_per + k * CHUNK
            pltpu.sync_copy(acc.at[pl.ds(r0, CHUNK)], buf_v)
            pltpu.sync_copy(buf_v, agg_hbm.at[pl.ds(c * Npad + r0, CHUNK)])

    return sc_agg


def _embed_body(x_ref, w_ref, b_ref, out_ref):
    h = jnp.dot(x_ref[...], w_ref[...], preferred_element_type=jnp.float32)
    h = h + b_ref[...]
    out_ref[0] = h[:, :32]
    out_ref[1] = h[:, 32:]


def _tc_embed(x, W_node, b_node, Nb):
    N, D = x.shape
    H = W_node.shape[1]
    grid = (N // Nb,)
    return pl.pallas_call(
        _embed_body,
        grid=grid,
        in_specs=[
            pl.BlockSpec((Nb, D), lambda i: (i, 0)),
            pl.BlockSpec((D, H), lambda i: (0, 0)),
            pl.BlockSpec((1, H), lambda i: (0, 0)),
        ],
        out_specs=pl.BlockSpec((2, Nb, H // 2), lambda i: (0, i, 0)),
        out_shape=jax.ShapeDtypeStruct((2, N, H // 2), jnp.float32),
    )(x, W_node, b_node.reshape(1, H))


def _update_body(h_ref, a_ref, w_ref, b_ref, s_ref, out_ref):
    h = jnp.concatenate([h_ref[0], h_ref[1]], axis=1)
    a = jnp.concatenate([a_ref[0], a_ref[1]], axis=1)
    u = jnp.dot(s_ref[0, 0] * h + a, w_ref[...],
                preferred_element_type=jnp.float32) + b_ref[...]
    u = jnp.maximum(u, 0.0) + h
    out_ref[0] = u[:, :32]
    out_ref[1] = u[:, 32:]


def _tc_update(h2, agg2, W, b, eps, Nb):
    _, N, Hh = h2.shape
    H = 2 * Hh
    grid = (N // Nb,)
    blk = pl.BlockSpec((2, Nb, Hh), lambda i: (0, i, 0))
    return pl.pallas_call(
        _update_body,
        grid=grid,
        in_specs=[
            blk, blk,
            pl.BlockSpec((H, H), lambda i: (0, 0)),
            pl.BlockSpec((1, H), lambda i: (0, 0)),
            pl.BlockSpec((1, 1), lambda i: (0, 0)),
        ],
        out_specs=blk,
        out_shape=jax.ShapeDtypeStruct((2, N, Hh), jnp.float32),
    )(h2, agg2, W, b.reshape(1, H), (1.0 + eps).reshape(1, 1))


def _final_body(h_ref, a_ref, w_ref, b_ref, s_ref, batch_ref,
                w1_ref, b1_ref, w2_ref, b2_ref, out_ref,
                pooled, cnt, *, NG, nsteps):
    i = pl.program_id(0)

    @pl.when(i == 0)
    def _():
        pooled[...] = jnp.zeros_like(pooled)
        cnt[...] = jnp.zeros_like(cnt)

    h = jnp.concatenate([h_ref[0], h_ref[1]], axis=1)
    a = jnp.concatenate([a_ref[0], a_ref[1]], axis=1)
    u = jnp.dot(s_ref[0, 0] * h + a, w_ref[...],
                preferred_element_type=jnp.float32) + b_ref[...]
    u = jnp.maximum(u, 0.0) + h

    gids = lax.broadcasted_iota(jnp.int32, (1, NG), 1)
    P = (batch_ref[...] == gids).astype(jnp.float32)
    pooled[...] += lax.dot_general(P, u, (((0,), (0,)), ((), ())),
                                   preferred_element_type=jnp.float32)
    cnt[...] += jnp.sum(P, axis=0, keepdims=True)

    @pl.when(i == nsteps - 1)
    def _():
        mean = pooled[...] / jnp.maximum(cnt[...], 1.0).T
        r = jnp.maximum(
            jnp.dot(mean, w1_ref[...], preferred_element_type=jnp.float32)
            + b1_ref[...], 0.0)
        out_ref[...] = (jnp.dot(r, w2_ref[...],
                                preferred_element_type=jnp.float32)
                        + b2_ref[...])


def _tc_final(h2, agg2, W, b, eps, batch, W_d1, b_d1, W_d2, b_d2, NG, Nb):
    _, N, Hh = h2.shape
    H = 2 * Hh
    Hd = W_d1.shape[1]
    nsteps = N // Nb
    blk = pl.BlockSpec((2, Nb, Hh), lambda i: (0, i, 0))
    body = functools.partial(_final_body, NG=NG, nsteps=nsteps)
    return pl.pallas_call(
        body,
        grid=(nsteps,),
        in_specs=[
            blk, blk,
            pl.BlockSpec((H, H), lambda i: (0, 0)),
            pl.BlockSpec((1, H), lambda i: (0, 0)),
            pl.BlockSpec((1, 1), lambda i: (0, 0)),
            pl.BlockSpec((Nb, 1), lambda i: (i, 0)),
            pl.BlockSpec((H, Hd), lambda i: (0, 0)),
            pl.BlockSpec((1, Hd), lambda i: (0, 0)),
            pl.BlockSpec((Hd, 1), lambda i: (0, 0)),
            pl.BlockSpec((1, 1), lambda i: (0, 0)),
        ],
        out_specs=pl.BlockSpec((NG, 1), lambda i: (0, 0)),
        out_shape=jax.ShapeDtypeStruct((NG, 1), jnp.float32),
        scratch_shapes=[
            pltpu.VMEM((NG, H), jnp.float32),
            pltpu.VMEM((1, NG), jnp.float32),
        ],
    )(h2, agg2, W, b.reshape(1, H), (1.0 + eps).reshape(1, 1),
      batch.reshape(N, 1), W_d1, b_d1.reshape(1, Hd), W_d2,
      b_d2.reshape(1, 1))


def kernel(x, edge_dist, edge_index, batch,
           W_node, b_node, W_edge, b_edge,
           W_c1, b_c1, eps1, W_c2, b_c2, eps2, W_c3, b_c3, eps3,
           W_d1, b_d1, W_d2, b_d2):
    N, _ = x.shape
    E = edge_dist.shape[0]
    H = W_node.shape[1]
    Hh = H // 2
    NG = 64
    Nb = 2000 if N % 2000 == 0 else N

    src = edge_index[0]
    dst = edge_index[1]
    wb = jnp.concatenate([W_edge[0], b_edge])

    sc_agg = _make_sc_aggregate(N, E, Hh)
    Npad = -(-N // (NS * 8)) * NS * 8

    h2 = _tc_embed(x, W_node, b_node, Nb)
    out = None
    for li, (W, b, eps) in enumerate(
            ((W_c1, b_c1, eps1), (W_c2, b_c2, eps2), (W_c3, b_c3, eps3))):
        agg = sc_agg(h2.reshape(2 * N, Hh), src, dst, edge_dist, wb)
        agg2 = agg.reshape(2, Npad, Hh)
        if li < 2:
            h2 = _tc_update(h2, agg2, W, b, eps, Nb)
        else:
            out = _tc_final(h2, agg2, W, b, eps, batch,
                            W_d1, b_d1, W_d2, b_d2, NG, Nb)
    return out


# speedup vs baseline: 1.3200x; 1.0446x over previous
import functools

import jax
import jax.numpy as jnp
from jax import lax
from jax.experimental import pallas as pl
from jax.experimental.pallas import tpu as pltpu
from jax.experimental.pallas import tpu_sc as plsc

NS = 16
NC = 2
LANES = 16


def _row_chunk(rows_per: int, max_rows: int = 640) -> int:
    for d in range(-(-rows_per // max_rows), rows_per + 1):
        if rows_per % d == 0:
            return rows_per // d
    return 1


def _make_sc_aggregate(N: int, E: int, Hh: int):
    assert E % NS == 0 and Hh == 32
    Eper = E // NS
    K = min(320, Eper)
    assert Eper % 8 == 0
    nblocks = -(-Eper // K)
    overlap = nblocks * K - Eper
    Npad = -(-N // (NS * 8)) * NS * 8
    rows_per = Npad // NS
    CHUNK = _row_chunk(rows_per, max_rows=256)
    assert CHUNK % 8 == 0
    nz = rows_per // CHUNK

    mesh = plsc.VectorSubcoreMesh(core_axis_name="c", subcore_axis_name="s",
                                  num_cores=NC, num_subcores=NS)

    @functools.partial(
        pl.kernel,
        out_type=jax.ShapeDtypeStruct((NC * Npad, Hh), jnp.float32),
        mesh=mesh,
        compiler_params=pltpu.CompilerParams(use_tc_tiling_on_sc=False),
        scratch_types=[
            pltpu.VMEM_SHARED((Npad, Hh), jnp.float32),
            pltpu.VMEM((K,), jnp.int32),
            pltpu.VMEM((K,), jnp.int32),
            pltpu.VMEM((K,), jnp.int32),
            pltpu.VMEM((K,), jnp.int32),
            pltpu.VMEM((K,), jnp.float32),
            pltpu.VMEM((K,), jnp.float32),
            pltpu.VMEM((K, Hh), jnp.float32),
            pltpu.VMEM((K, Hh), jnp.float32),
            pltpu.VMEM((K,), jnp.int32),
            pltpu.VMEM((K,), jnp.int32),
            pltpu.VMEM((2 * Hh,), jnp.float32),
            pltpu.VMEM((CHUNK, Hh), jnp.float32),
            pltpu.SemaphoreType.DMA,
            pltpu.SemaphoreType.DMA,
            pltpu.SemaphoreType.DMA,
            pltpu.SemaphoreType.DMA,
            pltpu.SemaphoreType.DMA,
            pltpu.SemaphoreType.DMA,
        ],
    )
    def sc_agg(h2_hbm, src_hbm, dst_hbm, dist_hbm, wb_hbm, agg_hbm,
               acc, idx0, idx1, dsx0, dsx1, dis0, dis1, msg0, msg1,
               snp0, snp1, wb_v, buf_v, sl0, sl1, sg0, sg1, ss0, ss1):
        IDX = (idx0, idx1)
        DSX = (dsx0, dsx1)
        DIS = (dis0, dis1)
        MSG = (msg0, msg1)
        SNP = (snp0, snp1)
        SL = (sl0, sl1)
        SG = (sg0, sg1)
        SS = (ss0, ss1)
        c = lax.axis_index("c")
        s = lax.axis_index("s")

        pltpu.sync_copy(wb_hbm.at[pl.ds(c * Hh, Hh)], wb_v.at[pl.ds(0, Hh)])
        pltpu.sync_copy(wb_hbm.at[pl.ds(NC * Hh + c * Hh, Hh)],
                        wb_v.at[pl.ds(Hh, Hh)])
        w_lo = wb_v[pl.ds(0, LANES)]
        w_hi = wb_v[pl.ds(LANES, LANES)]
        b_lo = wb_v[pl.ds(Hh, LANES)]
        b_hi = wb_v[pl.ds(Hh + LANES, LANES)]

        zvec = jnp.zeros((LANES,), jnp.float32)

        def zrow(r, _):
            buf_v[r, pl.ds(0, LANES)] = zvec
            buf_v[r, pl.ds(LANES, LANES)] = zvec
            return 0

        lax.fori_loop(0, CHUNK, zrow, 0)
        for k in range(nz):
            pltpu.sync_copy(buf_v, acc.at[pl.ds(s * rows_per + k * CHUNK, CHUNK)])
        plsc.subcore_barrier()

        row_off = c * N

        def base_of(b):
            return s * Eper + jnp.minimum(b * K, Eper - K)

        def start_loads(p, b):
            base = base_of(b)
            pltpu.async_copy(src_hbm.at[pl.ds(base, K)], IDX[p], SL[p])
            pltpu.async_copy(dst_hbm.at[pl.ds(base, K)], DSX[p], SL[p])
            pltpu.async_copy(dist_hbm.at[pl.ds(base, K)], DIS[p], SL[p])

        def wait_loads(p):
            pltpu.make_async_copy(src_hbm.at[pl.ds(0, K)], IDX[p], SL[p]).wait()
            pltpu.make_async_copy(dst_hbm.at[pl.ds(0, K)], DSX[p], SL[p]).wait()
            pltpu.make_async_copy(dist_hbm.at[pl.ds(0, K)], DIS[p], SL[p]).wait()

        def add_off(p):
            for i in range(K // LANES):
                sl = pl.ds(i * LANES, LANES)
                IDX[p][sl] = IDX[p][sl] + row_off

        def start_gather(p):
            pltpu.async_copy(h2_hbm.at[IDX[p]], MSG[p], SG[p])

        def wait_gather(p):
            pltpu.make_async_copy(h2_hbm.at[IDX[p]], MSG[p], SG[p]).wait()

        def start_scatter(p):
            for i in range(K // LANES):
                sl = pl.ds(i * LANES, LANES)
                SNP[p][sl] = DSX[p][sl]
            pltpu.async_copy(MSG[p], acc.at[SNP[p]], SS[p], add=True)

        def wait_scatter(p):
            pltpu.make_async_copy(MSG[p], acc.at[SNP[p]], SS[p]).wait()

        def compute(p, tail):
            msg_v, dist_v = MSG[p], DIS[p]

            def edge_grp(i, _):
                dvec = dist_v[pl.ds(i * LANES, LANES)]
                for j in range(LANES):
                    d = dvec[j]
                    r = i * LANES + j
                    lo = msg_v[r, pl.ds(0, LANES)]
                    hi = msg_v[r, pl.ds(LANES, LANES)]
                    msg_v[r, pl.ds(0, LANES)] = jnp.maximum(
                        lo + d * w_lo + b_lo, 0.0)
                    msg_v[r, pl.ds(LANES, LANES)] = jnp.maximum(
                        hi + d * w_hi + b_hi, 0.0)
                return 0

            lax.fori_loop(0, K // LANES, edge_grp, 0)
            if tail and overlap:
                for j in range(overlap):
                    msg_v[j, pl.ds(0, LANES)] = zvec
                    msg_v[j, pl.ds(LANES, LANES)] = zvec

        if nblocks < 4:
            def block(b, _):
                start_loads(0, b)
                wait_loads(0)
                add_off(0)
                start_gather(0)
                wait_gather(0)
                compute(0, False)

                @pl.when(b == nblocks - 1)
                def _():
                    if overlap:
                        for j in range(overlap):
                            msg0[j, pl.ds(0, LANES)] = zvec
                            msg0[j, pl.ds(LANES, LANES)] = zvec

                start_scatter(0)
                wait_scatter(0)
                return 0

            lax.fori_loop(0, nblocks, block, 0)
        else:
            M = (nblocks - 2) // 2 if nblocks % 2 == 0 else (nblocks - 3) // 2
            R = nblocks - 2 * M

            start_loads(0, 0)
            wait_loads(0)
            add_off(0)
            start_gather(0)
            start_loads(1, 1)

            def pair(g, _):
                wait_loads(1)
                add_off(1)

                @pl.when(g > 0)
                def _():
                    wait_scatter(1)

                start_gather(1)
                wait_gather(0)
                compute(0, False)
                start_scatter(0)
                start_loads(0, 2 * g + 2)
                wait_gather(1)
                wait_loads(0)
                add_off(0)
                wait_scatter(0)
                start_gather(0)
                start_loads(1, 2 * g + 3)
                compute(1, False)
                start_scatter(1)
                return 0

            lax.fori_loop(0, M, pair, 0)

            wait_gather(0)
            wait_loads(1)
            add_off(1)
            wait_scatter(1)
            start_gather(1)
            if R == 3:
                start_loads(0, nblocks - 1)
            compute(0, False)
            start_scatter(0)
            wait_gather(1)
            if R == 2:
                compute(1, True)
                start_scatter(1)
                wait_scatter(0)
                wait_scatter(1)
            else:
                wait_loads(0)
                add_off(0)
                wait_scatter(0)
                start_gather(0)
                compute(1, False)
                start_scatter(1)
                wait_gather(0)
                compute(0, True)
                start_scatter(0)
                wait_scatter(1)
                wait_scatter(0)

        plsc.subcore_barrier()

        for k in range(nz):
            r0 = s * rows_per + k * CHUNK
            pltpu.sync_copy(acc.at[pl.ds(r0, CHUNK)], buf_v)
            pltpu.sync_copy(buf_v, agg_hbm.at[pl.ds(c * Npad + r0, CHUNK)])

    return sc_agg


def _embed_body(x_ref, w_ref, b_ref, out_ref):
    h = jnp.dot(x_ref[...], w_ref[...], preferred_element_type=jnp.float32)
    h = h + b_ref[...]
    out_ref[0] = h[:, :32]
    out_ref[1] = h[:, 32:]


def _tc_embed(x, W_node, b_node, Nb):
    N, D = x.shape
    H = W_node.shape[1]
    grid = (N // Nb,)
    return pl.pallas_call(
        _embed_body,
        grid=grid,
        in_specs=[
            pl.BlockSpec((Nb, D), lambda i: (i, 0)),
            pl.BlockSpec((D, H), lambda i: (0, 0)),
            pl.BlockSpec((1, H), lambda i: (0, 0)),
        ],
        out_specs=pl.BlockSpec((2, Nb, H // 2), lambda i: (0, i, 0)),
        out_shape=jax.ShapeDtypeStruct((2, N, H // 2), jnp.float32),
    )(x, W_node, b_node.reshape(1, H))


def _update_body(h_ref, a_ref, w_ref, b_ref, s_ref, out_ref):
    h = jnp.concatenate([h_ref[0], h_ref[1]], axis=1)
    a = jnp.concatenate([a_ref[0], a_ref[1]], axis=1)
    u = jnp.dot(s_ref[0, 0] * h + a, w_ref[...],
                preferred_element_type=jnp.float32) + b_ref[...]
    u = jnp.maximum(u, 0.0) + h
    out_ref[0] = u[:, :32]
    out_ref[1] = u[:, 32:]


def _tc_update(h2, agg2, W, b, eps, Nb):
    _, N, Hh = h2.shape
    H = 2 * Hh
    grid = (N // Nb,)
    blk = pl.BlockSpec((2, Nb, Hh), lambda i: (0, i, 0))
    return pl.pallas_call(
        _update_body,
        grid=grid,
        in_specs=[
            blk, blk,
            pl.BlockSpec((H, H), lambda i: (0, 0)),
            pl.BlockSpec((1, H), lambda i: (0, 0)),
            pl.BlockSpec((1, 1), lambda i: (0, 0)),
        ],
        out_specs=blk,
        out_shape=jax.ShapeDtypeStruct((2, N, Hh), jnp.float32),
    )(h2, agg2, W, b.reshape(1, H), (1.0 + eps).reshape(1, 1))


def _final_body(h_ref, a_ref, w_ref, b_ref, s_ref, batch_ref,
                w1_ref, b1_ref, w2_ref, b2_ref, out_ref,
                pooled, cnt, *, NG, nsteps):
    i = pl.program_id(0)

    @pl.when(i == 0)
    def _():
        pooled[...] = jnp.zeros_like(pooled)
        cnt[...] = jnp.zeros_like(cnt)

    h = jnp.concatenate([h_ref[0], h_ref[1]], axis=1)
    a = jnp.concatenate([a_ref[0], a_ref[1]], axis=1)
    u = jnp.dot(s_ref[0, 0] * h + a, w_ref[...],
                preferred_element_type=jnp.float32) + b_ref[...]
    u = jnp.maximum(u, 0.0) + h

    gids = lax.broadcasted_iota(jnp.int32, (1, NG), 1)
    P = (batch_ref[...] == gids).astype(jnp.float32)
    pooled[...] += lax.dot_general(P, u, (((0,), (0,)), ((), ())),
                                   preferred_element_type=jnp.float32)
    cnt[...] += jnp.sum(P, axis=0, keepdims=True)

    @pl.when(i == nsteps - 1)
    def _():
        mean = pooled[...] / jnp.maximum(cnt[...], 1.0).T
        r = jnp.maximum(
            jnp.dot(mean, w1_ref[...], preferred_element_type=jnp.float32)
            + b1_ref[...], 0.0)
        out_ref[...] = (jnp.dot(r, w2_ref[...],
                                preferred_element_type=jnp.float32)
                        + b2_ref[...])


def _tc_final(h2, agg2, W, b, eps, batch, W_d1, b_d1, W_d2, b_d2, NG, Nb):
    _, N, Hh = h2.shape
    H = 2 * Hh
    Hd = W_d1.shape[1]
    nsteps = N // Nb
    blk = pl.BlockSpec((2, Nb, Hh), lambda i: (0, i, 0))
    body = functools.partial(_final_body, NG=NG, nsteps=nsteps)
    return pl.pallas_call(
        body,
        grid=(nsteps,),
        in_specs=[
            blk, blk,
            pl.BlockSpec((H, H), lambda i: (0, 0)),
            pl.BlockSpec((1, H), lambda i: (0, 0)),
            pl.BlockSpec((1, 1), lambda i: (0, 0)),
            pl.BlockSpec((Nb, 1), lambda i: (i, 0)),
            pl.BlockSpec((H, Hd), lambda i: (0, 0)),
            pl.BlockSpec((1, Hd), lambda i: (0, 0)),
            pl.BlockSpec((Hd, 1), lambda i: (0, 0)),
            pl.BlockSpec((1, 1), lambda i: (0, 0)),
        ],
        out_specs=pl.BlockSpec((NG, 1), lambda i: (0, 0)),
        out_shape=jax.ShapeDtypeStruct((NG, 1), jnp.float32),
        scratch_shapes=[
            pltpu.VMEM((NG, H), jnp.float32),
            pltpu.VMEM((1, NG), jnp.float32),
        ],
    )(h2, agg2, W, b.reshape(1, H), (1.0 + eps).reshape(1, 1),
      batch.reshape(N, 1), W_d1, b_d1.reshape(1, Hd), W_d2,
      b_d2.reshape(1, 1))


def kernel(x, edge_dist, edge_index, batch,
           W_node, b_node, W_edge, b_edge,
           W_c1, b_c1, eps1, W_c2, b_c2, eps2, W_c3, b_c3, eps3,
           W_d1, b_d1, W_d2, b_d2):
    N, _ = x.shape
    E = edge_dist.shape[0]
    H = W_node.shape[1]
    Hh = H // 2
    NG = 64
    Nb = 2000 if N % 2000 == 0 else N

    src = edge_index[0]
    dst = edge_index[1]
    wb = jnp.concatenate([W_edge[0], b_edge])

    sc_agg = _make_sc_aggregate(N, E, Hh)
    Npad = -(-N // (NS * 8)) * NS * 8

    h2 = _tc_embed(x, W_node, b_node, Nb)
    out = None
    for li, (W, b, eps) in enumerate(
            ((W_c1, b_c1, eps1), (W_c2, b_c2, eps2), (W_c3, b_c3, eps3))):
        agg = sc_agg(h2.reshape(2 * N, Hh), src, dst, edge_dist, wb)
        agg2 = agg.reshape(2, Npad, Hh)
        if li < 2:
            h2 = _tc_update(h2, agg2, W, b, eps, Nb)
        else:
            out = _tc_final(h2, agg2, W, b, eps, batch,
                            W_d1, b_d1, W_d2, b_d2, NG, Nb)
    return out


# speedup vs baseline: 1.3545x; 1.0261x over previous
import functools

import jax
import jax.numpy as jnp
from jax import lax
from jax.experimental import pallas as pl
from jax.experimental.pallas import tpu as pltpu
from jax.experimental.pallas import tpu_sc as plsc

NS = 16
NC = 2
LANES = 16


def _row_chunk(rows_per: int, max_rows: int = 640) -> int:
    for d in range(-(-rows_per // max_rows), rows_per + 1):
        if rows_per % d == 0:
            return rows_per // d
    return 1


def _make_sc_aggregate(N: int, E: int, Hh: int):
    assert E % NS == 0 and Hh == 32
    Eper = E // NS
    K = min(320, Eper)
    assert Eper % 8 == 0
    nblocks = -(-Eper // K)
    overlap = nblocks * K - Eper
    Npad = -(-N // (NS * 8)) * NS * 8
    rows_per = Npad // NS
    CHUNK = _row_chunk(rows_per, max_rows=256)
    assert CHUNK % 8 == 0
    nz = rows_per // CHUNK

    mesh = plsc.VectorSubcoreMesh(core_axis_name="c", subcore_axis_name="s",
                                  num_cores=NC, num_subcores=NS)

    @functools.partial(
        pl.kernel,
        out_type=jax.ShapeDtypeStruct((NC, Npad, Hh), jnp.float32),
        mesh=mesh,
        compiler_params=pltpu.CompilerParams(use_tc_tiling_on_sc=False),
        scratch_types=[
            pltpu.VMEM_SHARED((Npad, Hh), jnp.float32),
            pltpu.VMEM((K,), jnp.int32),
            pltpu.VMEM((K,), jnp.int32),
            pltpu.VMEM((K,), jnp.int32),
            pltpu.VMEM((K,), jnp.int32),
            pltpu.VMEM((K,), jnp.float32),
            pltpu.VMEM((K,), jnp.float32),
            pltpu.VMEM((K, Hh), jnp.float32),
            pltpu.VMEM((K, Hh), jnp.float32),
            pltpu.VMEM((K,), jnp.int32),
            pltpu.VMEM((K,), jnp.int32),
            pltpu.VMEM((2 * Hh,), jnp.float32),
            pltpu.VMEM((CHUNK, Hh), jnp.float32),
            pltpu.SemaphoreType.DMA,
            pltpu.SemaphoreType.DMA,
            pltpu.SemaphoreType.DMA,
            pltpu.SemaphoreType.DMA,
            pltpu.SemaphoreType.DMA,
            pltpu.SemaphoreType.DMA,
        ],
    )
    def sc_agg(h3_hbm, ei_hbm, dist_hbm, wb_hbm, agg_hbm,
               acc, idx0, idx1, dsx0, dsx1, dis0, dis1, msg0, msg1,
               snp0, snp1, wb_v, buf_v, sl0, sl1, sg0, sg1, ss0, ss1):
        IDX = (idx0, idx1)
        DSX = (dsx0, dsx1)
        DIS = (dis0, dis1)
        MSG = (msg0, msg1)
        SNP = (snp0, snp1)
        SL = (sl0, sl1)
        SG = (sg0, sg1)
        SS = (ss0, ss1)
        c = lax.axis_index("c")
        s = lax.axis_index("s")

        pltpu.sync_copy(wb_hbm.at[pl.ds(c * Hh, Hh)], wb_v.at[pl.ds(0, Hh)])
        pltpu.sync_copy(wb_hbm.at[pl.ds(NC * Hh + c * Hh, Hh)],
                        wb_v.at[pl.ds(Hh, Hh)])
        w_lo = wb_v[pl.ds(0, LANES)]
        w_hi = wb_v[pl.ds(LANES, LANES)]
        b_lo = wb_v[pl.ds(Hh, LANES)]
        b_hi = wb_v[pl.ds(Hh + LANES, LANES)]

        zvec = jnp.zeros((LANES,), jnp.float32)

        def zrow(r, _):
            buf_v[r, pl.ds(0, LANES)] = zvec
            buf_v[r, pl.ds(LANES, LANES)] = zvec
            return 0

        lax.fori_loop(0, CHUNK, zrow, 0)
        for k in range(nz):
            pltpu.sync_copy(buf_v, acc.at[pl.ds(s * rows_per + k * CHUNK, CHUNK)])
        plsc.subcore_barrier()

        h2_hbm = h3_hbm.at[c]

        def base_of(b):
            return s * Eper + jnp.minimum(b * K, Eper - K)

        def start_loads(p, b):
            base = base_of(b)
            pltpu.async_copy(ei_hbm.at[0, pl.ds(base, K)], IDX[p], SL[p])
            pltpu.async_copy(ei_hbm.at[1, pl.ds(base, K)], DSX[p], SL[p])
            pltpu.async_copy(dist_hbm.at[pl.ds(base, K)], DIS[p], SL[p])

        def wait_loads(p):
            pltpu.make_async_copy(ei_hbm.at[0, pl.ds(0, K)], IDX[p], SL[p]).wait()
            pltpu.make_async_copy(ei_hbm.at[1, pl.ds(0, K)], DSX[p], SL[p]).wait()
            pltpu.make_async_copy(dist_hbm.at[pl.ds(0, K)], DIS[p], SL[p]).wait()

        def add_off(p):
            pass

        def start_gather(p):
            pltpu.async_copy(h2_hbm.at[IDX[p]], MSG[p], SG[p])

        def wait_gather(p):
            pltpu.make_async_copy(h2_hbm.at[IDX[p]], MSG[p], SG[p]).wait()

        def start_scatter(p):
            for i in range(K // LANES):
                sl = pl.ds(i * LANES, LANES)
                SNP[p][sl] = DSX[p][sl]
            pltpu.async_copy(MSG[p], acc.at[SNP[p]], SS[p], add=True)

        def wait_scatter(p):
            pltpu.make_async_copy(MSG[p], acc.at[SNP[p]], SS[p]).wait()

        def compute(p, tail):
            msg_v, dist_v = MSG[p], DIS[p]

            def edge_grp(i, _):
                dvec = dist_v[pl.ds(i * LANES, LANES)]
                for j in range(LANES):
                    d = dvec[j]
                    r = i * LANES + j
                    lo = msg_v[r, pl.ds(0, LANES)]
                    hi = msg_v[r, pl.ds(LANES, LANES)]
                    msg_v[r, pl.ds(0, LANES)] = jnp.maximum(
                        lo + d * w_lo + b_lo, 0.0)
                    msg_v[r, pl.ds(LANES, LANES)] = jnp.maximum(
                        hi + d * w_hi + b_hi, 0.0)
                return 0

            lax.fori_loop(0, K // LANES, edge_grp, 0)
            if tail and overlap:
                for j in range(overlap):
                    msg_v[j, pl.ds(0, LANES)] = zvec
                    msg_v[j, pl.ds(LANES, LANES)] = zvec

        if nblocks < 4:
            def block(b, _):
                start_loads(0, b)
                wait_loads(0)
                add_off(0)
                start_gather(0)
                wait_gather(0)
                compute(0, False)

                @pl.when(b == nblocks - 1)
                def _():
                    if overlap:
                        for j in range(overlap):
                            msg0[j, pl.ds(0, LANES)] = zvec
                            msg0[j, pl.ds(LANES, LANES)] = zvec

                start_scatter(0)
                wait_scatter(0)
                return 0

            lax.fori_loop(0, nblocks, block, 0)
        else:
            M = (nblocks - 2) // 2 if nblocks % 2 == 0 else (nblocks - 3) // 2
            R = nblocks - 2 * M

            start_loads(0, 0)
            wait_loads(0)
            add_off(0)
            start_gather(0)
            start_loads(1, 1)

            def pair(g, _):
                wait_loads(1)
                add_off(1)

                @pl.when(g > 0)
                def _():
                    wait_scatter(1)

                start_gather(1)
                wait_gather(0)
                compute(0, False)
                start_scatter(0)
                start_loads(0, 2 * g + 2)
                wait_gather(1)
                wait_loads(0)
                add_off(0)
                wait_scatter(0)
                start_gather(0)
                start_loads(1, 2 * g + 3)
                compute(1, False)
                start_scatter(1)
                return 0

            lax.fori_loop(0, M, pair, 0)

            wait_gather(0)
            wait_loads(1)
            add_off(1)
            wait_scatter(1)
            start_gather(1)
            if R == 3:
                start_loads(0, nblocks - 1)
            compute(0, False)
            start_scatter(0)
            wait_gather(1)
            if R == 2:
                compute(1, True)
                start_scatter(1)
                wait_scatter(0)
                wait_scatter(1)
            else:
                wait_loads(0)
                add_off(0)
                wait_scatter(0)
                start_gather(0)
                compute(1, False)
                start_scatter(1)
                wait_gather(0)
                compute(0, True)
                start_scatter(0)
                wait_scatter(1)
                wait_scatter(0)

        plsc.subcore_barrier()

        for k in range(nz):
            r0 = s * rows_per + k * CHUNK
            pltpu.sync_copy(acc.at[pl.ds(r0, CHUNK)], buf_v)
            pltpu.sync_copy(buf_v, agg_hbm.at[c].at[pl.ds(r0, CHUNK)])

    return sc_agg


def _embed_body(x_ref, w_ref, b_ref, out_ref):
    h = jnp.dot(x_ref[...], w_ref[...], preferred_element_type=jnp.float32)
    h = h + b_ref[...]
    out_ref[0] = h[:, :32]
    out_ref[1] = h[:, 32:]


def _tc_embed(x, W_node, b_node, Nb):
    N, D = x.shape
    H = W_node.shape[1]
    grid = (N // Nb,)
    return pl.pallas_call(
        _embed_body,
        grid=grid,
        in_specs=[
            pl.BlockSpec((Nb, D), lambda i: (i, 0)),
            pl.BlockSpec((D, H), lambda i: (0, 0)),
            pl.BlockSpec((1, H), lambda i: (0, 0)),
        ],
        out_specs=pl.BlockSpec((2, Nb, H // 2), lambda i: (0, i, 0)),
        out_shape=jax.ShapeDtypeStruct((2, N, H // 2), jnp.float32),
    )(x, W_node, b_node.reshape(1, H))


def _update_body(h_ref, a_ref, w_ref, b_ref, s_ref, out_ref):
    h = jnp.concatenate([h_ref[0], h_ref[1]], axis=1)
    a = jnp.concatenate([a_ref[0], a_ref[1]], axis=1)
    u = jnp.dot(s_ref[0, 0] * h + a, w_ref[...],
                preferred_element_type=jnp.float32) + b_ref[...]
    u = jnp.maximum(u, 0.0) + h
    out_ref[0] = u[:, :32]
    out_ref[1] = u[:, 32:]


def _tc_update(h2, agg2, W, b, eps, Nb):
    _, N, Hh = h2.shape
    H = 2 * Hh
    grid = (N // Nb,)
    blk = pl.BlockSpec((2, Nb, Hh), lambda i: (0, i, 0))
    return pl.pallas_call(
        _update_body,
        grid=grid,
        in_specs=[
            blk, blk,
            pl.BlockSpec((H, H), lambda i: (0, 0)),
            pl.BlockSpec((1, H), lambda i: (0, 0)),
            pl.BlockSpec((1, 1), lambda i: (0, 0)),
        ],
        out_specs=blk,
        out_shape=jax.ShapeDtypeStruct((2, N, Hh), jnp.float32),
    )(h2, agg2, W, b.reshape(1, H), (1.0 + eps).reshape(1, 1))


def _final_body(h_ref, a_ref, w_ref, b_ref, s_ref, batch_ref,
                w1_ref, b1_ref, w2_ref, b2_ref, out_ref,
                pooled, cnt, *, NG, nsteps):
    i = pl.program_id(0)

    @pl.when(i == 0)
    def _():
        pooled[...] = jnp.zeros_like(pooled)
        cnt[...] = jnp.zeros_like(cnt)

    h = jnp.concatenate([h_ref[0], h_ref[1]], axis=1)
    a = jnp.concatenate([a_ref[0], a_ref[1]], axis=1)
    u = jnp.dot(s_ref[0, 0] * h + a, w_ref[...],
                preferred_element_type=jnp.float32) + b_ref[...]
    u = jnp.maximum(u, 0.0) + h

    gids = lax.broadcasted_iota(jnp.int32, (1, NG), 1)
    P = (batch_ref[...] == gids).astype(jnp.float32)
    pooled[...] += lax.dot_general(P, u, (((0,), (0,)), ((), ())),
                                   preferred_element_type=jnp.float32)
    cnt[...] += jnp.sum(P, axis=0, keepdims=True)

    @pl.when(i == nsteps - 1)
    def _():
        mean = pooled[...] / jnp.maximum(cnt[...], 1.0).T
        r = jnp.maximum(
            jnp.dot(mean, w1_ref[...], preferred_element_type=jnp.float32)
            + b1_ref[...], 0.0)
        out_ref[...] = (jnp.dot(r, w2_ref[...],
                                preferred_element_type=jnp.float32)
                        + b2_ref[...])


def _tc_final(h2, agg2, W, b, eps, batch, W_d1, b_d1, W_d2, b_d2, NG, Nb):
    _, N, Hh = h2.shape
    H = 2 * Hh
    Hd = W_d1.shape[1]
    nsteps = N // Nb
    blk = pl.BlockSpec((2, Nb, Hh), lambda i: (0, i, 0))
    body = functools.partial(_final_body, NG=NG, nsteps=nsteps)
    return pl.pallas_call(
        body,
        grid=(nsteps,),
        in_specs=[
            blk, blk,
            pl.BlockSpec((H, H), lambda i: (0, 0)),
            pl.BlockSpec((1, H), lambda i: (0, 0)),
            pl.BlockSpec((1, 1), lambda i: (0, 0)),
            pl.BlockSpec((Nb, 1), lambda i: (i, 0)),
            pl.BlockSpec((H, Hd), lambda i: (0, 0)),
            pl.BlockSpec((1, Hd), lambda i: (0, 0)),
            pl.BlockSpec((Hd, 1), lambda i: (0, 0)),
            pl.BlockSpec((1, 1), lambda i: (0, 0)),
        ],
        out_specs=pl.BlockSpec((NG, 1), lambda i: (0, 0)),
        out_shape=jax.ShapeDtypeStruct((NG, 1), jnp.float32),
        scratch_shapes=[
            pltpu.VMEM((NG, H), jnp.float32),
            pltpu.VMEM((1, NG), jnp.float32),
        ],
    )(h2, agg2, W, b.reshape(1, H), (1.0 + eps).reshape(1, 1),
      batch.reshape(N, 1), W_d1, b_d1.reshape(1, Hd), W_d2,
      b_d2.reshape(1, 1))


def kernel(x, edge_dist, edge_index, batch,
           W_node, b_node, W_edge, b_edge,
           W_c1, b_c1, eps1, W_c2, b_c2, eps2, W_c3, b_c3, eps3,
           W_d1, b_d1, W_d2, b_d2):
    N, _ = x.shape
    E = edge_dist.shape[0]
    H = W_node.shape[1]
    Hh = H // 2
    NG = 64
    Nb = 2000 if N % 2000 == 0 else N

    wb = jnp.concatenate([W_edge[0], b_edge])

    sc_agg = _make_sc_aggregate(N, E, Hh)

    h2 = _tc_embed(x, W_node, b_node, Nb)
    out = None
    for li, (W, b, eps) in enumerate(
            ((W_c1, b_c1, eps1), (W_c2, b_c2, eps2), (W_c3, b_c3, eps3))):
        agg2 = sc_agg(h2, edge_index, edge_dist, wb)
        if li < 2:
            h2 = _tc_update(h2, agg2, W, b, eps, Nb)
        else:
            out = _tc_final(h2, agg2, W, b, eps, batch,
                            W_d1, b_d1, W_d2, b_d2, NG, Nb)
    return out


# speedup vs baseline: 1.3957x; 1.0304x over previous
import functools

import jax
import jax.numpy as jnp
from jax import lax
from jax.experimental import pallas as pl
from jax.experimental.pallas import tpu as pltpu
from jax.experimental.pallas import tpu_sc as plsc

NS = 16
NC = 2
LANES = 16


def _row_chunk(rows_per: int, max_rows: int = 640) -> int:
    for d in range(-(-rows_per // max_rows), rows_per + 1):
        if rows_per % d == 0:
            return rows_per // d
    return 1


def _make_sc_aggregate(N: int, E: int, Hh: int):
    assert E % NS == 0 and Hh == 32
    Eper = E // NS
    K = min(384, Eper)
    assert Eper % 8 == 0
    nblocks = -(-Eper // K)
    overlap = nblocks * K - Eper
    Npad = -(-N // (NS * 8)) * NS * 8
    rows_per = Npad // NS

    mesh = plsc.VectorSubcoreMesh(core_axis_name="c", subcore_axis_name="s",
                                  num_cores=NC, num_subcores=NS)

    @functools.partial(
        pl.kernel,
        out_type=jax.ShapeDtypeStruct((NC, Npad, Hh), jnp.float32),
        mesh=mesh,
        compiler_params=pltpu.CompilerParams(use_tc_tiling_on_sc=False),
        scratch_types=[
            pltpu.VMEM_SHARED((Npad, Hh), jnp.float32),
            pltpu.VMEM((K,), jnp.int32),
            pltpu.VMEM((K,), jnp.int32),
            pltpu.VMEM((K,), jnp.int32),
            pltpu.VMEM((K,), jnp.int32),
            pltpu.VMEM((K,), jnp.float32),
            pltpu.VMEM((K,), jnp.float32),
            pltpu.VMEM((K, Hh), jnp.float32),
            pltpu.VMEM((K, Hh), jnp.float32),
            pltpu.VMEM((K,), jnp.int32),
            pltpu.VMEM((K,), jnp.int32),
            pltpu.VMEM((2 * Hh,), jnp.float32),
            pltpu.SemaphoreType.DMA,
            pltpu.SemaphoreType.DMA,
            pltpu.SemaphoreType.DMA,
            pltpu.SemaphoreType.DMA,
            pltpu.SemaphoreType.DMA,
            pltpu.SemaphoreType.DMA,
        ],
    )
    def sc_agg(h3_hbm, ei_hbm, dist_hbm, wb_hbm, agg_hbm,
               acc, idx0, idx1, dsx0, dsx1, dis0, dis1, msg0, msg1,
               snp0, snp1, wb_v, sl0, sl1, sg0, sg1, ss0, ss1):
        IDX = (idx0, idx1)
        DSX = (dsx0, dsx1)
        DIS = (dis0, dis1)
        MSG = (msg0, msg1)
        SNP = (snp0, snp1)
        SL = (sl0, sl1)
        SG = (sg0, sg1)
        SS = (ss0, ss1)
        c = lax.axis_index("c")
        s = lax.axis_index("s")

        pltpu.sync_copy(wb_hbm.at[pl.ds(c * Hh, Hh)], wb_v.at[pl.ds(0, Hh)])
        pltpu.sync_copy(wb_hbm.at[pl.ds(NC * Hh + c * Hh, Hh)],
                        wb_v.at[pl.ds(Hh, Hh)])
        w_lo = wb_v[pl.ds(0, LANES)]
        w_hi = wb_v[pl.ds(LANES, LANES)]
        b_lo = wb_v[pl.ds(Hh, LANES)]
        b_hi = wb_v[pl.ds(Hh + LANES, LANES)]

        zvec = jnp.zeros((LANES,), jnp.float32)
        r_base = s * rows_per
        ZC = _row_chunk(rows_per, K)

        def zrow(r, _):
            msg0[r, pl.ds(0, LANES)] = zvec
            msg0[r, pl.ds(LANES, LANES)] = zvec
            return 0

        lax.fori_loop(0, ZC, zrow, 0)

        def zcp(i, _):
            pltpu.sync_copy(msg0.at[pl.ds(0, ZC)],
                            acc.at[pl.ds(r_base + i * ZC, ZC)])
            return 0

        lax.fori_loop(0, rows_per // ZC, zcp, 0)
        plsc.subcore_barrier()

        h2_hbm = h3_hbm.at[c]

        def base_of(b):
            return s * Eper + jnp.minimum(b * K, Eper - K)

        def start_loads(p, b):
            base = base_of(b)
            pltpu.async_copy(ei_hbm.at[0, pl.ds(base, K)], IDX[p], SL[p])
            pltpu.async_copy(ei_hbm.at[1, pl.ds(base, K)], DSX[p], SL[p])
            pltpu.async_copy(dist_hbm.at[pl.ds(base, K)], DIS[p], SL[p])

        def wait_loads(p):
            pltpu.make_async_copy(ei_hbm.at[0, pl.ds(0, K)], IDX[p], SL[p]).wait()
            pltpu.make_async_copy(ei_hbm.at[1, pl.ds(0, K)], DSX[p], SL[p]).wait()
            pltpu.make_async_copy(dist_hbm.at[pl.ds(0, K)], DIS[p], SL[p]).wait()

        def add_off(p):
            pass

        def start_gather(p):
            pltpu.async_copy(h2_hbm.at[IDX[p]], MSG[p], SG[p])

        def wait_gather(p):
            pltpu.make_async_copy(h2_hbm.at[IDX[p]], MSG[p], SG[p]).wait()

        def start_scatter(p):
            for i in range(K // LANES):
                sl = pl.ds(i * LANES, LANES)
                SNP[p][sl] = DSX[p][sl]
            pltpu.async_copy(MSG[p], acc.at[SNP[p]], SS[p], add=True)

        def wait_scatter(p):
            pltpu.make_async_copy(MSG[p], acc.at[SNP[p]], SS[p]).wait()

        def compute(p, tail):
            msg_v, dist_v = MSG[p], DIS[p]

            def edge_grp(i, _):
                dvec = dist_v[pl.ds(i * LANES, LANES)]
                for j in range(LANES):
                    d = dvec[j]
                    r = i * LANES + j
                    lo = msg_v[r, pl.ds(0, LANES)]
                    hi = msg_v[r, pl.ds(LANES, LANES)]
                    msg_v[r, pl.ds(0, LANES)] = jnp.maximum(
                        lo + d * w_lo + b_lo, 0.0)
                    msg_v[r, pl.ds(LANES, LANES)] = jnp.maximum(
                        hi + d * w_hi + b_hi, 0.0)
                return 0

            lax.fori_loop(0, K // LANES, edge_grp, 0)
            if tail and overlap:
                for j in range(overlap):
                    msg_v[j, pl.ds(0, LANES)] = zvec
                    msg_v[j, pl.ds(LANES, LANES)] = zvec

        if nblocks < 4:
            def block(b, _):
                start_loads(0, b)
                wait_loads(0)
                add_off(0)
                start_gather(0)
                wait_gather(0)
                compute(0, False)

                @pl.when(b == nblocks - 1)
                def _():
                    if overlap:
                        for j in range(overlap):
                            msg0[j, pl.ds(0, LANES)] = zvec
                            msg0[j, pl.ds(LANES, LANES)] = zvec

                start_scatter(0)
                wait_scatter(0)
                return 0

            lax.fori_loop(0, nblocks, block, 0)
        else:
            M = (nblocks - 2) // 2 if nblocks % 2 == 0 else (nblocks - 3) // 2
            R = nblocks - 2 * M

            start_loads(0, 0)
            wait_loads(0)
            add_off(0)
            start_gather(0)
            start_loads(1, 1)

            def pair(g, _):
                wait_loads(1)
                add_off(1)

                @pl.when(g > 0)
                def _():
                    wait_scatter(1)

                start_gather(1)
                wait_gather(0)
                compute(0, False)
                start_scatter(0)
                start_loads(0, 2 * g + 2)
                wait_gather(1)
                wait_loads(0)
                add_off(0)
                wait_scatter(0)
                start_gather(0)
                start_loads(1, 2 * g + 3)
                compute(1, False)
                start_scatter(1)
                return 0

            lax.fori_loop(0, M, pair, 0)

            wait_gather(0)
            wait_loads(1)
            add_off(1)
            wait_scatter(1)
            start_gather(1)
            if R == 3:
                start_loads(0, nblocks - 1)
            compute(0, False)
            start_scatter(0)
            wait_gather(1)
            if R == 2:
                compute(1, True)
                start_scatter(1)
                wait_scatter(0)
                wait_scatter(1)
            else:
                wait_loads(0)
                add_off(0)
                wait_scatter(0)
                start_gather(0)
                compute(1, False)
                start_scatter(1)
                wait_gather(0)
                compute(0, True)
                start_scatter(0)
                wait_scatter(1)
                wait_scatter(0)

        plsc.subcore_barrier()

        pltpu.sync_copy(acc.at[pl.ds(r_base, rows_per)],
                        agg_hbm.at[c].at[pl.ds(r_base, rows_per)])

    return sc_agg


def _embed_body(x_ref, w_ref, b_ref, out_ref):
    h = jnp.dot(x_ref[...], w_ref[...], preferred_element_type=jnp.float32)
    h = h + b_ref[...]
    out_ref[0] = h[:, :32]
    out_ref[1] = h[:, 32:]


def _tc_embed(x, W_node, b_node, Nb):
    N, D = x.shape
    H = W_node.shape[1]
    grid = (N // Nb,)
    return pl.pallas_call(
        _embed_body,
        grid=grid,
        in_specs=[
            pl.BlockSpec((Nb, D), lambda i: (i, 0)),
            pl.BlockSpec((D, H), lambda i: (0, 0)),
            pl.BlockSpec((1, H), lambda i: (0, 0)),
        ],
        out_specs=pl.BlockSpec((2, Nb, H // 2), lambda i: (0, i, 0)),
        out_shape=jax.ShapeDtypeStruct((2, N, H // 2), jnp.float32),
    )(x, W_node, b_node.reshape(1, H))


def _update_body(h_ref, a_ref, w_ref, b_ref, s_ref, out_ref):
    h = jnp.concatenate([h_ref[0], h_ref[1]], axis=1)
    a = jnp.concatenate([a_ref[0], a_ref[1]], axis=1)
    u = jnp.dot(s_ref[0, 0] * h + a, w_ref[...],
                preferred_element_type=jnp.float32) + b_ref[...]
    u = jnp.maximum(u, 0.0) + h
    out_ref[0] = u[:, :32]
    out_ref[1] = u[:, 32:]


def _tc_update(h2, agg2, W, b, eps, Nb):
    _, N, Hh = h2.shape
    H = 2 * Hh
    grid = (N // Nb,)
    blk = pl.BlockSpec((2, Nb, Hh), lambda i: (0, i, 0))
    return pl.pallas_call(
        _update_body,
        grid=grid,
        in_specs=[
            blk, blk,
            pl.BlockSpec((H, H), lambda i: (0, 0)),
            pl.BlockSpec((1, H), lambda i: (0, 0)),
            pl.BlockSpec((1, 1), lambda i: (0, 0)),
        ],
        out_specs=blk,
        out_shape=jax.ShapeDtypeStruct((2, N, Hh), jnp.float32),
    )(h2, agg2, W, b.reshape(1, H), (1.0 + eps).reshape(1, 1))


def _final_body(h_ref, a_ref, w_ref, b_ref, s_ref, batch_ref,
                w1_ref, b1_ref, w2_ref, b2_ref, out_ref,
                pooled, cnt, *, NG, nsteps):
    i = pl.program_id(0)

    @pl.when(i == 0)
    def _():
        pooled[...] = jnp.zeros_like(pooled)
        cnt[...] = jnp.zeros_like(cnt)

    h = jnp.concatenate([h_ref[0], h_ref[1]], axis=1)
    a = jnp.concatenate([a_ref[0], a_ref[1]], axis=1)
    u = jnp.dot(s_ref[0, 0] * h + a, w_ref[...],
                preferred_element_type=jnp.float32) + b_ref[...]
    u = jnp.maximum(u, 0.0) + h

    gids = lax.broadcasted_iota(jnp.int32, (1, NG), 1)
    P = (batch_ref[...] == gids).astype(jnp.float32)
    pooled[...] += lax.dot_general(P, u, (((0,), (0,)), ((), ())),
                                   preferred_element_type=jnp.float32)
    cnt[...] += jnp.sum(P, axis=0, keepdims=True)

    @pl.when(i == nsteps - 1)
    def _():
        mean = pooled[...] / jnp.maximum(cnt[...], 1.0).T
        r = jnp.maximum(
            jnp.dot(mean, w1_ref[...], preferred_element_type=jnp.float32)
            + b1_ref[...], 0.0)
        out_ref[...] = (jnp.dot(r, w2_ref[...],
                                preferred_element_type=jnp.float32)
                        + b2_ref[...])


def _tc_final(h2, agg2, W, b, eps, batch, W_d1, b_d1, W_d2, b_d2, NG, Nb):
    _, N, Hh = h2.shape
    H = 2 * Hh
    Hd = W_d1.shape[1]
    nsteps = N // Nb
    blk = pl.BlockSpec((2, Nb, Hh), lambda i: (0, i, 0))
    body = functools.partial(_final_body, NG=NG, nsteps=nsteps)
    return pl.pallas_call(
        body,
        grid=(nsteps,),
        in_specs=[
            blk, blk,
            pl.BlockSpec((H, H), lambda i: (0, 0)),
            pl.BlockSpec((1, H), lambda i: (0, 0)),
            pl.BlockSpec((1, 1), lambda i: (0, 0)),
            pl.BlockSpec((Nb, 1), lambda i: (i, 0)),
            pl.BlockSpec((H, Hd), lambda i: (0, 0)),
            pl.BlockSpec((1, Hd), lambda i: (0, 0)),
            pl.BlockSpec((Hd, 1), lambda i: (0, 0)),
            pl.BlockSpec((1, 1), lambda i: (0, 0)),
        ],
        out_specs=pl.BlockSpec((NG, 1), lambda i: (0, 0)),
        out_shape=jax.ShapeDtypeStruct((NG, 1), jnp.float32),
        scratch_shapes=[
            pltpu.VMEM((NG, H), jnp.float32),
            pltpu.VMEM((1, NG), jnp.float32),
        ],
    )(h2, agg2, W, b.reshape(1, H), (1.0 + eps).reshape(1, 1),
      batch.reshape(N, 1), W_d1, b_d1.reshape(1, Hd), W_d2,
      b_d2.reshape(1, 1))


def kernel(x, edge_dist, edge_index, batch,
           W_node, b_node, W_edge, b_edge,
           W_c1, b_c1, eps1, W_c2, b_c2, eps2, W_c3, b_c3, eps3,
           W_d1, b_d1, W_d2, b_d2):
    N, _ = x.shape
    E = edge_dist.shape[0]
    H = W_node.shape[1]
    Hh = H // 2
    NG = 64
    Nb = 2000 if N % 2000 == 0 else N

    wb = jnp.concatenate([W_edge[0], b_edge])

    sc_agg = _make_sc_aggregate(N, E, Hh)

    h2 = _tc_embed(x, W_node, b_node, Nb)
    out = None
    for li, (W, b, eps) in enumerate(
            ((W_c1, b_c1, eps1), (W_c2, b_c2, eps2), (W_c3, b_c3, eps3))):
        agg2 = sc_agg(h2, edge_index, edge_dist, wb)
        if li < 2:
            h2 = _tc_update(h2, agg2, W, b, eps, Nb)
        else:
            out = _tc_final(h2, agg2, W, b, eps, batch,
                            W_d1, b_d1, W_d2, b_d2, NG, Nb)
    return out


# speedup vs baseline: 1.4076x; 1.0085x over previous
import functools

import jax
import jax.numpy as jnp
from jax import lax
from jax.experimental import pallas as pl
from jax.experimental.pallas import tpu as pltpu
from jax.experimental.pallas import tpu_sc as plsc

NS = 16
NC = 2
LANES = 16


def _row_chunk(rows_per: int, max_rows: int = 640) -> int:
    for d in range(-(-rows_per // max_rows), rows_per + 1):
        if rows_per % d == 0 and (rows_per // d) % 8 == 0:
            return rows_per // d
    return 8


def _make_sc_aggregate(N: int, E: int, Hh: int):
    assert E % NS == 0 and Hh == 32
    Eper = E // NS
    K = min(416, Eper)
    assert Eper % 8 == 0
    nblocks = -(-Eper // K)
    overlap = nblocks * K - Eper
    Npad = -(-N // (NS * 8)) * NS * 8
    rows_per = Npad // NS

    mesh = plsc.VectorSubcoreMesh(core_axis_name="c", subcore_axis_name="s",
                                  num_cores=NC, num_subcores=NS)

    @functools.partial(
        pl.kernel,
        out_type=jax.ShapeDtypeStruct((NC, Npad, Hh), jnp.float32),
        mesh=mesh,
        compiler_params=pltpu.CompilerParams(use_tc_tiling_on_sc=False),
        scratch_types=[
            pltpu.VMEM_SHARED((Npad, Hh), jnp.float32),
            pltpu.VMEM((K,), jnp.int32),
            pltpu.VMEM((K,), jnp.int32),
            pltpu.VMEM((K,), jnp.int32),
            pltpu.VMEM((K,), jnp.int32),
            pltpu.VMEM((K,), jnp.float32),
            pltpu.VMEM((K,), jnp.float32),
            pltpu.VMEM((K, Hh), jnp.float32),
            pltpu.VMEM((K, Hh), jnp.float32),
            pltpu.VMEM((K,), jnp.int32),
            pltpu.VMEM((K,), jnp.int32),
            pltpu.VMEM((2 * Hh,), jnp.float32),
            pltpu.SemaphoreType.DMA,
            pltpu.SemaphoreType.DMA,
            pltpu.SemaphoreType.DMA,
            pltpu.SemaphoreType.DMA,
            pltpu.SemaphoreType.DMA,
            pltpu.SemaphoreType.DMA,
        ],
    )
    def sc_agg(h3_hbm, ei_hbm, dist_hbm, wb_hbm, agg_hbm,
               acc, idx0, idx1, dsx0, dsx1, dis0, dis1, msg0, msg1,
               snp0, snp1, wb_v, sl0, sl1, sg0, sg1, ss0, ss1):
        IDX = (idx0, idx1)
        DSX = (dsx0, dsx1)
        DIS = (dis0, dis1)
        MSG = (msg0, msg1)
        SNP = (snp0, snp1)
        SL = (sl0, sl1)
        SG = (sg0, sg1)
        SS = (ss0, ss1)
        c = lax.axis_index("c")
        s = lax.axis_index("s")

        pltpu.sync_copy(wb_hbm.at[pl.ds(c * Hh, Hh)], wb_v.at[pl.ds(0, Hh)])
        pltpu.sync_copy(wb_hbm.at[pl.ds(NC * Hh + c * Hh, Hh)],
                        wb_v.at[pl.ds(Hh, Hh)])
        w_lo = wb_v[pl.ds(0, LANES)]
        w_hi = wb_v[pl.ds(LANES, LANES)]
        b_lo = wb_v[pl.ds(Hh, LANES)]
        b_hi = wb_v[pl.ds(Hh + LANES, LANES)]

        zvec = jnp.zeros((LANES,), jnp.float32)
        r_base = s * rows_per
        ZC = _row_chunk(rows_per, K)

        def zrow(r, _):
            msg0[r, pl.ds(0, LANES)] = zvec
            msg0[r, pl.ds(LANES, LANES)] = zvec
            return 0

        lax.fori_loop(0, ZC, zrow, 0)

        def zcp(i, _):
            pltpu.sync_copy(msg0.at[pl.ds(0, ZC)],
                            acc.at[pl.ds(r_base + i * ZC, ZC)])
            return 0

        lax.fori_loop(0, rows_per // ZC, zcp, 0)
        plsc.subcore_barrier()

        h2_hbm = h3_hbm.at[c]

        def base_of(b):
            return s * Eper + jnp.minimum(b * K, Eper - K)

        def start_loads(p, b):
            base = base_of(b)
            pltpu.async_copy(ei_hbm.at[0, pl.ds(base, K)], IDX[p], SL[p])
            pltpu.async_copy(ei_hbm.at[1, pl.ds(base, K)], DSX[p], SL[p])
            pltpu.async_copy(dist_hbm.at[pl.ds(base, K)], DIS[p], SL[p])

        def wait_loads(p):
            pltpu.make_async_copy(ei_hbm.at[0, pl.ds(0, K)], IDX[p], SL[p]).wait()
            pltpu.make_async_copy(ei_hbm.at[1, pl.ds(0, K)], DSX[p], SL[p]).wait()
            pltpu.make_async_copy(dist_hbm.at[pl.ds(0, K)], DIS[p], SL[p]).wait()

        def add_off(p):
            pass

        def start_gather(p):
            pltpu.async_copy(h2_hbm.at[IDX[p]], MSG[p], SG[p])

        def wait_gather(p):
            pltpu.make_async_copy(h2_hbm.at[IDX[p]], MSG[p], SG[p]).wait()

        def start_scatter(p):
            for i in range(K // LANES):
                sl = pl.ds(i * LANES, LANES)
                SNP[p][sl] = DSX[p][sl]
            pltpu.async_copy(MSG[p], acc.at[SNP[p]], SS[p], add=True)

        def wait_scatter(p):
            pltpu.make_async_copy(MSG[p], acc.at[SNP[p]], SS[p]).wait()

        def compute(p, tail):
            msg_v, dist_v = MSG[p], DIS[p]

            def edge_grp(i, _):
                dvec = dist_v[pl.ds(i * LANES, LANES)]
                for j in range(LANES):
                    d = dvec[j]
                    r = i * LANES + j
                    lo = msg_v[r, pl.ds(0, LANES)]
                    hi = msg_v[r, pl.ds(LANES, LANES)]
                    msg_v[r, pl.ds(0, LANES)] = jnp.maximum(
                        lo + d * w_lo + b_lo, 0.0)
                    msg_v[r, pl.ds(LANES, LANES)] = jnp.maximum(
                        hi + d * w_hi + b_hi, 0.0)
                return 0

            lax.fori_loop(0, K // LANES, edge_grp, 0)
            if tail and overlap:
                for j in range(overlap):
                    msg_v[j, pl.ds(0, LANES)] = zvec
                    msg_v[j, pl.ds(LANES, LANES)] = zvec

        if nblocks < 4:
            def block(b, _):
                start_loads(0, b)
                wait_loads(0)
                add_off(0)
                start_gather(0)
                wait_gather(0)
                compute(0, False)

                @pl.when(b == nblocks - 1)
                def _():
                    if overlap:
                        for j in range(overlap):
                            msg0[j, pl.ds(0, LANES)] = zvec
                            msg0[j, pl.ds(LANES, LANES)] = zvec

                start_scatter(0)
                wait_scatter(0)
                return 0

            lax.fori_loop(0, nblocks, block, 0)
        else:
            M = (nblocks - 2) // 2 if nblocks % 2 == 0 else (nblocks - 3) // 2
            R = nblocks - 2 * M

            start_loads(0, 0)
            wait_loads(0)
            add_off(0)
            start_gather(0)
            start_loads(1, 1)

            def pair(g, _):
                wait_loads(1)
                add_off(1)

                @pl.when(g > 0)
                def _():
                    wait_scatter(1)

                start_gather(1)
                wait_gather(0)
                compute(0, False)
                start_scatter(0)
                start_loads(0, 2 * g + 2)
                wait_gather(1)
                wait_loads(0)
                add_off(0)
                wait_scatter(0)
                start_gather(0)
                start_loads(1, 2 * g + 3)
                compute(1, False)
                start_scatter(1)
                return 0

            lax.fori_loop(0, M, pair, 0)

            wait_gather(0)
            wait_loads(1)
            add_off(1)
            wait_scatter(1)
            start_gather(1)
            if R == 3:
                start_loads(0, nblocks - 1)
            compute(0, False)
            start_scatter(0)
            wait_gather(1)
            if R == 2:
                compute(1, True)
                start_scatter(1)
                wait_scatter(0)
                wait_scatter(1)
            else:
                wait_loads(0)
                add_off(0)
                wait_scatter(0)
                start_gather(0)
                compute(1, False)
                start_scatter(1)
                wait_gather(0)
                compute(0, True)
                start_scatter(0)
                wait_scatter(1)
                wait_scatter(0)

        plsc.subcore_barrier()

        pltpu.sync_copy(acc.at[pl.ds(r_base, rows_per)],
                        agg_hbm.at[c].at[pl.ds(r_base, rows_per)])

    return sc_agg


def _embed_body(x_ref, w_ref, b_ref, out_ref):
    h = jnp.dot(x_ref[...], w_ref[...], preferred_element_type=jnp.float32)
    h = h + b_ref[...]
    out_ref[0] = h[:, :32]
    out_ref[1] = h[:, 32:]


def _tc_embed(x, W_node, b_node, Nb):
    N, D = x.shape
    H = W_node.shape[1]
    grid = (N // Nb,)
    return pl.pallas_call(
        _embed_body,
        grid=grid,
        in_specs=[
            pl.BlockSpec((Nb, D), lambda i: (i, 0)),
            pl.BlockSpec((D, H), lambda i: (0, 0)),
            pl.BlockSpec((1, H), lambda i: (0, 0)),
        ],
        out_specs=pl.BlockSpec((2, Nb, H // 2), lambda i: (0, i, 0)),
        out_shape=jax.ShapeDtypeStruct((2, N, H // 2), jnp.float32),
    )(x, W_node, b_node.reshape(1, H))


def _update_body(h_ref, a_ref, w_ref, b_ref, s_ref, out_ref):
    h = jnp.concatenate([h_ref[0], h_ref[1]], axis=1)
    a = jnp.concatenate([a_ref[0], a_ref[1]], axis=1)
    u = jnp.dot(s_ref[0, 0] * h + a, w_ref[...],
                preferred_element_type=jnp.float32) + b_ref[...]
    u = jnp.maximum(u, 0.0) + h
    out_ref[0] = u[:, :32]
    out_ref[1] = u[:, 32:]


def _tc_update(h2, agg2, W, b, eps, Nb):
    _, N, Hh = h2.shape
    H = 2 * Hh
    grid = (N // Nb,)
    blk = pl.BlockSpec((2, Nb, Hh), lambda i: (0, i, 0))
    return pl.pallas_call(
        _update_body,
        grid=grid,
        in_specs=[
            blk, blk,
            pl.BlockSpec((H, H), lambda i: (0, 0)),
            pl.BlockSpec((1, H), lambda i: (0, 0)),
            pl.BlockSpec((1, 1), lambda i: (0, 0)),
        ],
        out_specs=blk,
        out_shape=jax.ShapeDtypeStruct((2, N, Hh), jnp.float32),
    )(h2, agg2, W, b.reshape(1, H), (1.0 + eps).reshape(1, 1))


def _final_body(h_ref, a_ref, w_ref, b_ref, s_ref, batch_ref,
                w1_ref, b1_ref, w2_ref, b2_ref, out_ref,
                pooled, cnt, *, NG, nsteps):
    i = pl.program_id(0)

    @pl.when(i == 0)
    def _():
        pooled[...] = jnp.zeros_like(pooled)
        cnt[...] = jnp.zeros_like(cnt)

    h = jnp.concatenate([h_ref[0], h_ref[1]], axis=1)
    a = jnp.concatenate([a_ref[0], a_ref[1]], axis=1)
    u = jnp.dot(s_ref[0, 0] * h + a, w_ref[...],
                preferred_element_type=jnp.float32) + b_ref[...]
    u = jnp.maximum(u, 0.0) + h

    gids = lax.broadcasted_iota(jnp.int32, (1, NG), 1)
    P = (batch_ref[...] == gids).astype(jnp.float32)
    pooled[...] += lax.dot_general(P, u, (((0,), (0,)), ((), ())),
                                   preferred_element_type=jnp.float32)
    cnt[...] += jnp.sum(P, axis=0, keepdims=True)

    @pl.when(i == nsteps - 1)
    def _():
        mean = pooled[...] / jnp.maximum(cnt[...], 1.0).T
        r = jnp.maximum(
            jnp.dot(mean, w1_ref[...], preferred_element_type=jnp.float32)
            + b1_ref[...], 0.0)
        out_ref[...] = (jnp.dot(r, w2_ref[...],
                                preferred_element_type=jnp.float32)
                        + b2_ref[...])


def _tc_final(h2, agg2, W, b, eps, batch, W_d1, b_d1, W_d2, b_d2, NG, Nb):
    _, N, Hh = h2.shape
    H = 2 * Hh
    Hd = W_d1.shape[1]
    nsteps = N // Nb
    blk = pl.BlockSpec((2, Nb, Hh), lambda i: (0, i, 0))
    body = functools.partial(_final_body, NG=NG, nsteps=nsteps)
    return pl.pallas_call(
        body,
        grid=(nsteps,),
        in_specs=[
            blk, blk,
            pl.BlockSpec((H, H), lambda i: (0, 0)),
            pl.BlockSpec((1, H), lambda i: (0, 0)),
            pl.BlockSpec((1, 1), lambda i: (0, 0)),
            pl.BlockSpec((Nb, 1), lambda i: (i, 0)),
            pl.BlockSpec((H, Hd), lambda i: (0, 0)),
            pl.BlockSpec((1, Hd), lambda i: (0, 0)),
            pl.BlockSpec((Hd, 1), lambda i: (0, 0)),
            pl.BlockSpec((1, 1), lambda i: (0, 0)),
        ],
        out_specs=pl.BlockSpec((NG, 1), lambda i: (0, 0)),
        out_shape=jax.ShapeDtypeStruct((NG, 1), jnp.float32),
        scratch_shapes=[
            pltpu.VMEM((NG, H), jnp.float32),
            pltpu.VMEM((1, NG), jnp.float32),
        ],
    )(h2, agg2, W, b.reshape(1, H), (1.0 + eps).reshape(1, 1),
      batch.reshape(N, 1), W_d1, b_d1.reshape(1, Hd), W_d2,
      b_d2.reshape(1, 1))


def kernel(x, edge_dist, edge_index, batch,
           W_node, b_node, W_edge, b_edge,
           W_c1, b_c1, eps1, W_c2, b_c2, eps2, W_c3, b_c3, eps3,
           W_d1, b_d1, W_d2, b_d2):
    N, _ = x.shape
    E = edge_dist.shape[0]
    H = W_node.shape[1]
    Hh = H // 2
    NG = 64
    Nb = 2000 if N % 2000 == 0 else N

    wb = jnp.concatenate([W_edge[0], b_edge])

    sc_agg = _make_sc_aggregate(N, E, Hh)

    h2 = _tc_embed(x, W_node, b_node, Nb)
    out = None
    for li, (W, b, eps) in enumerate(
            ((W_c1, b_c1, eps1), (W_c2, b_c2, eps2), (W_c3, b_c3, eps3))):
        agg2 = sc_agg(h2, edge_index, edge_dist, wb)
        if li < 2:
            h2 = _tc_update(h2, agg2, W, b, eps, Nb)
        else:
            out = _tc_final(h2, agg2, W, b, eps, batch,
                            W_d1, b_d1, W_d2, b_d2, NG, Nb)
    return out
